# baseline (speedup 1.0000x reference)
import functools

import jax
import jax.numpy as jnp
from jax import lax
from jax.experimental import pallas as pl
from jax.experimental.pallas import tpu as pltpu
from jax.experimental.pallas import tpu_sc as plsc

N_NODES = 10000
N_EDGES = 320000
D = 128
HEADS = 4
C = 32

NUM_CORES = 2
NUM_TILES = 16
NW = NUM_CORES * NUM_TILES
BLK = 64
SB = 4
E_PER_W = 10240
E_REAL_W = N_EDGES // NW
N_BLKS = E_PER_W // BLK
N_SUPER = N_BLKS // SB
E_PAD = E_PER_W * NW
N_PAD = 10112
ROWS_PER_TILE = N_PAD // NUM_TILES

_HI_MASK = -65536


def _prep_body(x_ref, w1_ref, w2_ref, o_ref):
    a1 = lax.dot_general(
        x_ref[...], w1_ref[...], (((1,), (0,)), ((), ())),
        preferred_element_type=jnp.float32, precision=lax.Precision.HIGHEST)
    a2 = lax.dot_general(
        x_ref[...], w2_ref[...], (((1,), (0,)), ((), ())),
        preferred_element_type=jnp.float32, precision=lax.Precision.HIGHEST)
    b1 = lax.bitcast_convert_type(a1, jnp.int32)
    b2 = lax.bitcast_convert_type(a2, jnp.int32)
    r1 = (b1 + 0x8000) & _HI_MASK
    r2 = lax.shift_right_logical(b2 + 0x8000, 16)
    o_ref[...] = lax.transpose(r1 | r2, (1, 0))


_tc_prep = pl.pallas_call(
    _prep_body,
    out_shape=jax.ShapeDtypeStruct((HEADS, N_NODES), jnp.int32),
)


def _comb_body(p_ref, o_ref):
    o_ref[...] = p_ref[0, :N_NODES] + p_ref[1, :N_NODES]


_tc_combine = pl.pallas_call(
    _comb_body,
    out_shape=jax.ShapeDtypeStruct((N_NODES, D), jnp.float32),
)


def _sc_body(tab_hbm, sde_hbm, x_hbm, zeros_hbm, out_hbm,
             tab_v, idxew_v, xj_v, betat_v, shared_out, sem_x):
    c = lax.axis_index("c")
    s = lax.axis_index("s")
    tid = c * NUM_TILES + s
    edge0 = tid * E_PER_W

    pltpu.sync_copy(zeros_hbm, shared_out.at[pl.ds(s * ROWS_PER_TILE, ROWS_PER_TILE)])
    pltpu.sync_copy(tab_hbm, tab_v)
    plsc.subcore_barrier()

    def superblock(sb, carry):
        pltpu.sync_copy(sde_hbm.at[tid, sb], idxew_v)
        for b in range(SB):
            _block(b)
        return carry

    def _block(b):
        cx = pltpu.async_copy(x_hbm.at[idxew_v.at[1, b]], xj_v, sem_x)

        for g in range(BLK // 16):
            sl = pl.ds(g * 16, 16)
            s16 = idxew_v[0, b, sl]
            d16 = idxew_v[1, b, sl]
            w16 = plsc.bitcast(idxew_v[2, b, sl], jnp.float32)
            for h in range(HEADS):
                ws = plsc.load_gather(tab_v, [s16 + (h * N_NODES)])
                wd = plsc.load_gather(tab_v, [d16 + (h * N_NODES)])
                a1 = plsc.bitcast(ws & _HI_MASK, jnp.float32)
                a2 = plsc.bitcast(lax.shift_left(wd, 16), jnp.float32)
                beta = w16 / (1.0 + jnp.exp(-(a1 + a2)))
                betat_v[pl.ds(h * BLK + g * 16, 16)] = beta

        cx.wait()

        def edge8(t, carry2):
            e0 = t * 8
            bs = []
            for r in range(8):
                e_idx = jnp.full((16,), r, jnp.int32) + e0
                bs.append([
                    plsc.load_gather(betat_v, [e_idx + (h * BLK)])
                    for h in range(HEADS)
                ])
            for r in range(8):
                e = e0 + r
                for h in range(HEADS):
                    for k in range(C // 16):
                        fsl = pl.ds(h * C + k * 16, 16)
                        xj_v[e, fsl] = xj_v[e, fsl] * bs[r][h]
            return carry2

        lax.fori_loop(0, BLK // 8, edge8, 0)

        pltpu.sync_copy(xj_v, shared_out.at[idxew_v.at[0, b]], add=True)

    lax.fori_loop(0, N_SUPER, superblock, 0)
    plsc.subcore_barrier()

    rsl = pl.ds(s * ROWS_PER_TILE, ROWS_PER_TILE)
    pltpu.sync_copy(shared_out.at[rsl], out_hbm.at[c, rsl])


_sc_main = functools.partial(
    pl.kernel,
    out_type=jax.ShapeDtypeStruct((NUM_CORES, N_PAD, D), jnp.float32),
    mesh=plsc.VectorSubcoreMesh(core_axis_name="c", subcore_axis_name="s"),
    compiler_params=pltpu.CompilerParams(
        needs_layout_passes=False, use_tc_tiling_on_sc=False),
    scratch_types=[
        pltpu.VMEM((HEADS * N_NODES,), jnp.int32),
        pltpu.VMEM((3, SB, BLK), jnp.int32),
        pltpu.VMEM((BLK, D), jnp.float32),
        pltpu.VMEM((HEADS * BLK,), jnp.float32),
        pltpu.VMEM_SHARED((N_PAD, D), jnp.float32),
        pltpu.SemaphoreType.DMA,
    ],
)(_sc_body)


def kernel(x_tangent0, edge_index, edge_weight, W):
    src = edge_index[0].astype(jnp.int32)
    dst = edge_index[1].astype(jnp.int32)
    ew_bits = lax.bitcast_convert_type(edge_weight, jnp.int32)
    pad_cols = E_PER_W - E_REAL_W

    def _tile_layout(v):
        padded = jnp.pad(v.reshape(NW, E_REAL_W), ((0, 0), (0, pad_cols)))
        return padded.reshape(NW, N_SUPER, SB, BLK)

    sde = jnp.stack(
        [_tile_layout(src), _tile_layout(dst), _tile_layout(ew_bits)], axis=2)
    w1 = W[0, :C]
    w2 = W[0, C:]
    eye = jnp.eye(HEADS, dtype=jnp.float32)
    wm1 = jnp.kron(eye, w1[:, None])
    wm2 = jnp.kron(eye, w2[:, None])
    tab = _tc_prep(x_tangent0, wm1, wm2).reshape(-1)
    zeros = jnp.zeros((ROWS_PER_TILE, D), jnp.float32)
    partials = _sc_main(tab, sde, x_tangent0, zeros)
    return _tc_combine(partials)


# speedup vs baseline: 1.4430x; 1.4430x over previous
import functools

import jax
import jax.numpy as jnp
from jax import lax
from jax.experimental import pallas as pl
from jax.experimental.pallas import tpu as pltpu
from jax.experimental.pallas import tpu_sc as plsc

N_NODES = 10000
N_EDGES = 320000
D = 128
HEADS = 4
C = 32

NUM_CORES = 2
NUM_TILES = 16
NW = NUM_CORES * NUM_TILES
BLK = 64
E_PER_W = 10048
E_REAL_W = N_EDGES // NW
N_BLKS = E_PER_W // BLK
E_PAD = E_PER_W * NW
N_PAD = 10240
ROWS_PER_TILE = N_PAD // NUM_TILES

_HI_MASK = -65536


def _prep_body(x_ref, w1_ref, w2_ref, o_ref):
    a1 = lax.dot_general(
        x_ref[...], w1_ref[...], (((1,), (0,)), ((), ())),
        preferred_element_type=jnp.float32, precision=lax.Precision.HIGHEST)
    a2 = lax.dot_general(
        x_ref[...], w2_ref[...], (((1,), (0,)), ((), ())),
        preferred_element_type=jnp.float32, precision=lax.Precision.HIGHEST)
    b1 = lax.bitcast_convert_type(a1, jnp.int32)
    b2 = lax.bitcast_convert_type(a2, jnp.int32)
    r1 = (b1 + 0x8000) & _HI_MASK
    r2 = lax.shift_right_logical(b2 + 0x8000, 16)
    o_ref[...] = lax.transpose(r1 | r2, (1, 0))


_tc_prep = pl.pallas_call(
    _prep_body,
    out_shape=jax.ShapeDtypeStruct((HEADS, N_NODES), jnp.int32),
)


def _comb_body(p_ref, o_ref):
    o_ref[...] = p_ref[0, :N_NODES] + p_ref[1, :N_NODES]


_tc_combine = pl.pallas_call(
    _comb_body,
    out_shape=jax.ShapeDtypeStruct((N_NODES, D), jnp.float32),
)


def _sc_body(tab_hbm, sde_hbm, x_hbm, zeros_hbm, out_hbm,
             tab_v, idxew_v, xj_v, betat_v, shared_out, sem_x):
    c = lax.axis_index("c")
    s = lax.axis_index("s")
    tid = c * NUM_TILES + s
    edge0 = tid * E_PER_W

    pltpu.sync_copy(zeros_hbm, shared_out.at[pl.ds(s * ROWS_PER_TILE, ROWS_PER_TILE)])
    pltpu.sync_copy(tab_hbm, tab_v)
    plsc.subcore_barrier()

    def block(i, carry):
        base = edge0 + i * BLK
        pltpu.sync_copy(sde_hbm.at[:, pl.ds(base, BLK)], idxew_v)
        cx = pltpu.async_copy(x_hbm.at[idxew_v.at[1]], xj_v, sem_x)

        for g in range(BLK // 16):
            sl = pl.ds(g * 16, 16)
            s16 = idxew_v[0, sl]
            d16 = idxew_v[1, sl]
            w16 = plsc.bitcast(idxew_v[2, sl], jnp.float32)
            for h in range(HEADS):
                ws = plsc.load_gather(tab_v, [s16 + (h * N_NODES)])
                wd = plsc.load_gather(tab_v, [d16 + (h * N_NODES)])
                a1 = plsc.bitcast(ws & _HI_MASK, jnp.float32)
                a2 = plsc.bitcast(lax.shift_left(wd, 16), jnp.float32)
                beta = w16 / (1.0 + jnp.exp(-(a1 + a2)))
                betat_v[pl.ds(h * BLK + g * 16, 16)] = beta

        cx.wait()

        def edge8(t, carry2):
            e0 = t * 8
            bs = []
            for r in range(8):
                e_idx = jnp.full((16,), r, jnp.int32) + e0
                bs.append([
                    plsc.load_gather(betat_v, [e_idx + (h * BLK)])
                    for h in range(HEADS)
                ])
            for r in range(8):
                e = e0 + r
                for h in range(HEADS):
                    for k in range(C // 16):
                        fsl = pl.ds(h * C + k * 16, 16)
                        xj_v[e, fsl] = xj_v[e, fsl] * bs[r][h]
            return carry2

        lax.fori_loop(0, BLK // 8, edge8, 0)

        pltpu.sync_copy(xj_v, shared_out.at[idxew_v.at[0]], add=True)
        return carry

    lax.fori_loop(0, N_BLKS, block, 0)
    plsc.subcore_barrier()

    rsl = pl.ds(s * ROWS_PER_TILE, ROWS_PER_TILE)
    pltpu.sync_copy(shared_out.at[rsl], out_hbm.at[c, rsl])


_sc_main = functools.partial(
    pl.kernel,
    out_type=jax.ShapeDtypeStruct((NUM_CORES, N_PAD, D), jnp.float32),
    mesh=plsc.VectorSubcoreMesh(core_axis_name="c", subcore_axis_name="s"),
    compiler_params=pltpu.CompilerParams(
        needs_layout_passes=False, use_tc_tiling_on_sc=False),
    scratch_types=[
        pltpu.VMEM((HEADS * N_NODES,), jnp.int32),
        pltpu.VMEM((3, BLK), jnp.int32),
        pltpu.VMEM((BLK, D), jnp.float32),
        pltpu.VMEM((HEADS * BLK,), jnp.float32),
        pltpu.VMEM_SHARED((N_PAD, D), jnp.float32),
        pltpu.SemaphoreType.DMA,
    ],
)(_sc_body)


def kernel(x_tangent0, edge_index, edge_weight, W):
    src = edge_index[0].astype(jnp.int32)
    dst = edge_index[1].astype(jnp.int32)
    ew_bits = lax.bitcast_convert_type(edge_weight, jnp.int32)
    pad_cols = E_PER_W - E_REAL_W
    sde = jnp.stack([
        jnp.pad(src.reshape(NW, E_REAL_W), ((0, 0), (0, pad_cols))).reshape(-1),
        jnp.pad(dst.reshape(NW, E_REAL_W), ((0, 0), (0, pad_cols))).reshape(-1),
        jnp.pad(ew_bits.reshape(NW, E_REAL_W), ((0, 0), (0, pad_cols))).reshape(-1),
    ])
    w1 = W[0, :C]
    w2 = W[0, C:]
    eye = jnp.eye(HEADS, dtype=jnp.float32)
    wm1 = jnp.kron(eye, w1[:, None])
    wm2 = jnp.kron(eye, w2[:, None])
    tab = _tc_prep(x_tangent0, wm1, wm2).reshape(-1)
    zeros = jnp.zeros((ROWS_PER_TILE, D), jnp.float32)
    partials = _sc_main(tab, sde, x_tangent0, zeros)
    return _tc_combine(partials)
